# baseline (speedup 1.0000x reference)
import functools

import jax
import jax.numpy as jnp
from jax import lax
from jax.experimental import pallas as pl
from jax.experimental.pallas import tpu as pltpu
from jax.experimental.pallas import tpu_sc as plsc

BLK = 256
NW = 32


def _group_selector(n_groups, in_dim, dtype):
    gsz = in_dim // n_groups
    col_g = lax.broadcasted_iota(jnp.int32, (n_groups, in_dim), 1) // gsz
    row_g = lax.broadcasted_iota(jnp.int32, (n_groups, in_dim), 0)
    return (col_g == row_g).astype(dtype)


def _dequant_to(q_ref, z_ref, s_ref, w_ref):
    O, I = w_ref.shape
    n_g = s_ref.shape[-1]
    sel = _group_selector(n_g, I, jnp.bfloat16)
    s = s_ref[0]
    zs = z_ref[0].astype(jnp.float32) * s
    dims = (((1,), (0,)), ((), ()))
    s_full = lax.dot_general(s.astype(jnp.bfloat16), sel, dims,
                             preferred_element_type=jnp.float32)
    zs_full = lax.dot_general(zs.astype(jnp.bfloat16), sel, dims,
                              preferred_element_type=jnp.float32)
    w = q_ref[0].astype(jnp.float32) * s_full - zs_full
    w_ref[...] = w.astype(jnp.bfloat16)


def _plan_body(x_ref, rw_ref, pos0_ref, pos1_ref, w0_ref, w1_ref,
               bexp_ref, nused_ref, *, nb):
    E, T = rw_ref.shape[0], x_ref.shape[0]
    logits = lax.dot_general(rw_ref[...], x_ref[...],
                             (((1,), (1,)), ((), ())),
                             preferred_element_type=jnp.float32)
    s = jax.nn.sigmoid(logits)
    riota = lax.broadcasted_iota(jnp.int32, s.shape, 0)
    i1 = jnp.argmax(s, axis=0)
    m1 = jnp.max(s, axis=0)
    oh1 = riota == i1[None, :]
    s2 = jnp.where(oh1, -1.0, s)
    i2 = jnp.argmax(s2, axis=0)
    m2 = jnp.max(s2, axis=0)
    oh2 = riota == i2[None, :]
    denom = m1 + m2
    w0_ref[...] = (m1 / denom)[None, :]
    w1_ref[...] = (m2 / denom)[None, :]

    r_t = lax.broadcasted_iota(jnp.int32, (T, T), 0)
    c_t = lax.broadcasted_iota(jnp.int32, (T, T), 1)
    tri = (r_t <= c_t).astype(jnp.bfloat16)
    dims = (((1,), (0,)), ((), ()))
    cum1 = lax.dot_general(oh1.astype(jnp.bfloat16), tri, dims,
                           preferred_element_type=jnp.float32)
    cum2 = lax.dot_general(oh2.astype(jnp.bfloat16), tri, dims,
                           preferred_element_type=jnp.float32)
    cnt1 = cum1[:, T - 1:T]
    counts = cnt1 + cum2[:, T - 1:T]
    padded = jnp.floor((counts + (BLK - 1)) * (1.0 / BLK)) * BLK
    r_e = lax.broadcasted_iota(jnp.int32, (E, E), 0)
    c_e = lax.broadcasted_iota(jnp.int32, (E, E), 1)
    sl = (c_e < r_e).astype(jnp.float32)
    seg_base = lax.dot_general(sl, padded, dims,
                               precision=lax.Precision.HIGHEST,
                               preferred_element_type=jnp.float32)
    pos0 = jnp.sum(jnp.where(oh1, seg_base + cum1 - 1.0, 0.0), axis=0)
    pos1 = jnp.sum(jnp.where(oh2, seg_base + cnt1 + cum2 - 1.0, 0.0), axis=0)
    pos0_ref[...] = pos0[None, :].astype(jnp.int32)
    pos1_ref[...] = pos1[None, :].astype(jnp.int32)

    bstart = (lax.broadcasted_iota(jnp.int32, (E, nb), 1) * BLK).astype(jnp.float32)
    bexp = jnp.sum((seg_base <= bstart).astype(jnp.int32), axis=0) - 1
    bexp_ref[...] = bexp[None, :]
    total = seg_base[E - 1:E, :] + padded[E - 1:E, :]
    nused_ref[...] = (total * (1.0 / BLK)).astype(jnp.int32)


def _sc_dispatch_body(pos0_h, pos1_h, x_h, xs_h, p_v, rows_v, sem0, sem1,
                      *, T, P):
    wid = lax.axis_index("s") * 2 + lax.axis_index("c")
    tokens_per = T // NW
    tbase = wid * tokens_per
    pltpu.sync_copy(pos0_h.at[pl.ds(tbase, tokens_per)], p_v.at[0])
    pltpu.sync_copy(pos1_h.at[pl.ds(tbase, tokens_per)], p_v.at[1])
    pltpu.sync_copy(x_h.at[pl.ds(tbase, tokens_per)], rows_v)
    c0 = pltpu.async_copy(rows_v, xs_h.at[p_v.at[0]], sem0)
    c1 = pltpu.async_copy(rows_v, xs_h.at[p_v.at[1]], sem1)
    c0.wait()
    c1.wait()


def _moe_routed_body(bexp_ref, nused_ref, xs_ref,
                     gq, gz, gs, uq, uz, us, dq, dz, ds,
                     ys_ref, wg_s, wu_s, wd_s):
    b = pl.program_id(0)
    e = bexp_ref[b]
    prev = bexp_ref[jnp.maximum(b - 1, 0)]

    @pl.when((b == 0) | (e != prev))
    def _():
        _dequant_to(gq, gz, gs, wg_s)
        _dequant_to(uq, uz, us, wu_s)
        _dequant_to(dq, dz, ds, wd_s)

    @pl.when(b < nused_ref[0])
    def _():
        xb = xs_ref[...].astype(jnp.bfloat16)
        dims = (((1,), (1,)), ((), ()))
        g = lax.dot_general(xb, wg_s[...], dims, preferred_element_type=jnp.float32)
        u = lax.dot_general(xb, wu_s[...], dims, preferred_element_type=jnp.float32)
        h = (g * jax.nn.sigmoid(g) * u).astype(jnp.bfloat16)
        ys_ref[...] = lax.dot_general(h, wd_s[...], dims,
                                      preferred_element_type=jnp.float32)


def _sc_collect_body(pos0_h, pos1_h, ys_h, y0_h, y1_h,
                     p_v, b0, b1, sem0, sem1, *, T, D):
    wid = lax.axis_index("s") * 2 + lax.axis_index("c")
    tokens_per = T // NW
    tbase = wid * tokens_per
    pltpu.sync_copy(pos0_h.at[pl.ds(tbase, tokens_per)], p_v.at[0])
    pltpu.sync_copy(pos1_h.at[pl.ds(tbase, tokens_per)], p_v.at[1])
    rows = b0.shape[0]
    for c in range(tokens_per // rows):
        c0 = pltpu.async_copy(ys_h.at[p_v.at[0, pl.ds(c * rows, rows)]], b0, sem0)
        c1 = pltpu.async_copy(ys_h.at[p_v.at[1, pl.ds(c * rows, rows)]], b1, sem1)
        c0.wait()
        pltpu.sync_copy(b0, y0_h.at[pl.ds(tbase + c * rows, rows)])
        c1.wait()
        pltpu.sync_copy(b1, y1_h.at[pl.ds(tbase + c * rows, rows)])


def _shared_body(xb_ref, sg_ref, su_ref, sd_ref, out_ref):
    xb = xb_ref[...]
    dims = (((1,), (1,)), ((), ()))
    a = lax.dot_general(xb, sg_ref[...], dims, preferred_element_type=jnp.float32)
    b = lax.dot_general(xb, su_ref[...], dims, preferred_element_type=jnp.float32)
    h = (a * jax.nn.sigmoid(a) * b).astype(jnp.bfloat16)
    out_ref[...] = lax.dot_general(h, sd_ref[...], dims,
                                   preferred_element_type=jnp.float32)


def _combine_body(sh_ref, y0_ref, y1_ref, w0_ref, w1_ref, out_ref):
    out_ref[...] = (sh_ref[...] + y0_ref[...] * w0_ref[...]
                    + y1_ref[...] * w1_ref[...])


def kernel(hidden_states, router_w, gate_q, gate_z, gate_s, up_q, up_z, up_s,
           down_q, down_z, down_s, shared_gate_w, shared_up_w, shared_down_w):
    B, S, D = hidden_states.shape
    T = B * S
    E, FF, _ = gate_q.shape
    n_g_in = gate_s.shape[-1]
    n_g_ff = down_s.shape[-1]
    SFF = shared_gate_w.shape[0]
    NB = (2 * T) // BLK + E
    P = NB * BLK

    x = hidden_states.reshape(T, D)
    xb = x.astype(jnp.bfloat16)
    gq8 = gate_q.astype(jnp.int8)
    gz8 = gate_z.astype(jnp.int8)
    uq8 = up_q.astype(jnp.int8)
    uz8 = up_z.astype(jnp.int8)
    dq8 = down_q.astype(jnp.int8)
    dz8 = down_z.astype(jnp.int8)
    sgb = shared_gate_w.astype(jnp.bfloat16)
    sub = shared_up_w.astype(jnp.bfloat16)
    sdb = shared_down_w.astype(jnp.bfloat16)

    pos0, pos1, w0, w1, bexp, nused = pl.pallas_call(
        functools.partial(_plan_body, nb=NB),
        grid=(1,),
        in_specs=[
            pl.BlockSpec((T, D), lambda i: (0, 0)),
            pl.BlockSpec((E, D), lambda i: (0, 0)),
        ],
        out_specs=[
            pl.BlockSpec((1, T), lambda i: (0, 0)),
            pl.BlockSpec((1, T), lambda i: (0, 0)),
            pl.BlockSpec((1, T), lambda i: (0, 0)),
            pl.BlockSpec((1, T), lambda i: (0, 0)),
            pl.BlockSpec((1, NB), lambda i: (0, 0)),
            pl.BlockSpec((1, 1), lambda i: (0, 0)),
        ],
        out_shape=[
            jax.ShapeDtypeStruct((1, T), jnp.int32),
            jax.ShapeDtypeStruct((1, T), jnp.int32),
            jax.ShapeDtypeStruct((1, T), jnp.float32),
            jax.ShapeDtypeStruct((1, T), jnp.float32),
            jax.ShapeDtypeStruct((1, NB), jnp.int32),
            jax.ShapeDtypeStruct((1, 1), jnp.int32),
        ],
    )(x, router_w)
    pos0f = pos0.reshape(T)
    pos1f = pos1.reshape(T)
    w0f = w0.reshape(T)
    w1f = w1.reshape(T)

    mesh = plsc.VectorSubcoreMesh(core_axis_name="c", subcore_axis_name="s")
    xs = pl.kernel(
        functools.partial(_sc_dispatch_body, T=T, P=P),
        mesh=mesh,
        out_type=jax.ShapeDtypeStruct((P, D), jnp.float32),
        scratch_types=[
            pltpu.VMEM((2, T // NW), jnp.int32),
            pltpu.VMEM((T // NW, D), jnp.float32),
            pltpu.SemaphoreType.DMA,
            pltpu.SemaphoreType.DMA,
        ],
    )(pos0f, pos1f, x)

    w_spec = lambda shape: pl.BlockSpec(
        (1,) + shape, lambda b, be, nu: (be[b], 0, 0))
    ys = pl.pallas_call(
        _moe_routed_body,
        grid_spec=pltpu.PrefetchScalarGridSpec(
            num_scalar_prefetch=2,
            grid=(NB,),
            in_specs=[
                pl.BlockSpec((BLK, D), lambda b, be, nu: (b, 0)),
                w_spec((FF, D)), w_spec((FF, n_g_in)), w_spec((FF, n_g_in)),
                w_spec((FF, D)), w_spec((FF, n_g_in)), w_spec((FF, n_g_in)),
                w_spec((D, FF)), w_spec((D, n_g_ff)), w_spec((D, n_g_ff)),
            ],
            out_specs=pl.BlockSpec((BLK, D), lambda b, be, nu: (b, 0)),
            scratch_shapes=[
                pltpu.VMEM((FF, D), jnp.bfloat16),
                pltpu.VMEM((FF, D), jnp.bfloat16),
                pltpu.VMEM((D, FF), jnp.bfloat16),
            ],
        ),
        out_shape=jax.ShapeDtypeStruct((P, D), jnp.float32),
    )(bexp.reshape(NB), nused.reshape(1), xs,
      gq8, gz8, gate_s, uq8, uz8, up_s, dq8, dz8, down_s)

    y0, y1 = pl.kernel(
        functools.partial(_sc_collect_body, T=T, D=D),
        mesh=mesh,
        out_type=[jax.ShapeDtypeStruct((T, D), jnp.float32),
                  jax.ShapeDtypeStruct((T, D), jnp.float32)],
        scratch_types=[
            pltpu.VMEM((2, T // NW), jnp.int32),
            pltpu.VMEM((T // NW // 2, D), jnp.float32),
            pltpu.VMEM((T // NW // 2, D), jnp.float32),
            pltpu.SemaphoreType.DMA,
            pltpu.SemaphoreType.DMA,
        ],
    )(pos0f, pos1f, ys)

    blk_s = min(T, 256)
    shared_y = pl.pallas_call(
        _shared_body,
        grid=(T // blk_s,),
        in_specs=[
            pl.BlockSpec((blk_s, D), lambda m: (m, 0)),
            pl.BlockSpec((SFF, D), lambda m: (0, 0)),
            pl.BlockSpec((SFF, D), lambda m: (0, 0)),
            pl.BlockSpec((D, SFF), lambda m: (0, 0)),
        ],
        out_specs=pl.BlockSpec((blk_s, D), lambda m: (m, 0)),
        out_shape=jax.ShapeDtypeStruct((T, D), jnp.float32),
    )(xb, sgb, sub, sdb)

    blk_c = min(T, 512)
    out = pl.pallas_call(
        _combine_body,
        grid=(T // blk_c,),
        in_specs=[
            pl.BlockSpec((blk_c, D), lambda m: (m, 0)),
            pl.BlockSpec((blk_c, D), lambda m: (m, 0)),
            pl.BlockSpec((blk_c, D), lambda m: (m, 0)),
            pl.BlockSpec((blk_c, 1), lambda m: (m, 0)),
            pl.BlockSpec((blk_c, 1), lambda m: (m, 0)),
        ],
        out_specs=pl.BlockSpec((blk_c, D), lambda m: (m, 0)),
        out_shape=jax.ShapeDtypeStruct((T, D), jnp.float32),
    )(shared_y, y0, y1, w0f.reshape(T, 1), w1f.reshape(T, 1))

    return out.reshape(B, S, D)


# speedup vs baseline: 1.0469x; 1.0469x over previous
import functools

import jax
import jax.numpy as jnp
from jax import lax
from jax.experimental import pallas as pl
from jax.experimental.pallas import tpu as pltpu
from jax.experimental.pallas import tpu_sc as plsc

BLK = 256
NW = 32


def _group_selector(n_groups, in_dim, dtype):
    gsz = in_dim // n_groups
    col_g = lax.broadcasted_iota(jnp.int32, (n_groups, in_dim), 1) // gsz
    row_g = lax.broadcasted_iota(jnp.int32, (n_groups, in_dim), 0)
    return (col_g == row_g).astype(dtype)


def _dequant_to(q_ref, z_ref, s_ref, w_ref):
    O, I = w_ref.shape
    n_g = s_ref.shape[-1]
    sel = _group_selector(n_g, I, jnp.bfloat16)
    s = s_ref[0]
    zs = z_ref[0].astype(jnp.float32) * s
    dims = (((1,), (0,)), ((), ()))
    s_full = lax.dot_general(s.astype(jnp.bfloat16), sel, dims,
                             preferred_element_type=jnp.float32)
    zs_full = lax.dot_general(zs.astype(jnp.bfloat16), sel, dims,
                              preferred_element_type=jnp.float32)
    w = q_ref[0].astype(jnp.float32) * s_full - zs_full
    w_ref[...] = w.astype(jnp.bfloat16)


def _plan_body(x_ref, rw_ref, pos0_ref, pos1_ref, w0_ref, w1_ref,
               bexp_ref, nused_ref, *, nb):
    E, T = rw_ref.shape[0], x_ref.shape[0]
    logits = lax.dot_general(rw_ref[...], x_ref[...],
                             (((1,), (1,)), ((), ())),
                             preferred_element_type=jnp.float32)
    s = jax.nn.sigmoid(logits)
    riota = lax.broadcasted_iota(jnp.int32, s.shape, 0)
    i1 = jnp.argmax(s, axis=0)
    m1 = jnp.max(s, axis=0)
    oh1 = riota == i1[None, :]
    s2 = jnp.where(oh1, -1.0, s)
    i2 = jnp.argmax(s2, axis=0)
    m2 = jnp.max(s2, axis=0)
    oh2 = riota == i2[None, :]
    denom = m1 + m2
    w0_ref[...] = (m1 / denom)[None, :]
    w1_ref[...] = (m2 / denom)[None, :]

    r_t = lax.broadcasted_iota(jnp.int32, (T, T), 0)
    c_t = lax.broadcasted_iota(jnp.int32, (T, T), 1)
    tri = (r_t <= c_t).astype(jnp.bfloat16)
    dims = (((1,), (0,)), ((), ()))
    cum1 = lax.dot_general(oh1.astype(jnp.bfloat16), tri, dims,
                           preferred_element_type=jnp.float32)
    cum2 = lax.dot_general(oh2.astype(jnp.bfloat16), tri, dims,
                           preferred_element_type=jnp.float32)
    cnt1 = cum1[:, T - 1:T]
    counts = cnt1 + cum2[:, T - 1:T]
    padded = jnp.floor((counts + (BLK - 1)) * (1.0 / BLK)) * BLK
    r_e = lax.broadcasted_iota(jnp.int32, (E, E), 0)
    c_e = lax.broadcasted_iota(jnp.int32, (E, E), 1)
    sl = (c_e < r_e).astype(jnp.float32)
    seg_base = lax.dot_general(sl, padded, dims,
                               precision=lax.Precision.HIGHEST,
                               preferred_element_type=jnp.float32)
    pos0 = jnp.sum(jnp.where(oh1, seg_base + cum1 - 1.0, 0.0), axis=0)
    pos1 = jnp.sum(jnp.where(oh2, seg_base + cnt1 + cum2 - 1.0, 0.0), axis=0)
    pos0_ref[...] = pos0[None, :].astype(jnp.int32)
    pos1_ref[...] = pos1[None, :].astype(jnp.int32)

    bstart = (lax.broadcasted_iota(jnp.int32, (E, nb), 1) * BLK).astype(jnp.float32)
    bexp = jnp.sum((seg_base <= bstart).astype(jnp.int32), axis=0) - 1
    bexp_ref[...] = bexp[None, :]
    total = seg_base[E - 1:E, :] + padded[E - 1:E, :]
    nused_ref[...] = (total * (1.0 / BLK)).astype(jnp.int32)


def _sc_dispatch_body(pos0_h, pos1_h, x_h, xs_h, p_v, rows_v, sem0, sem1,
                      *, T, P):
    wid = lax.axis_index("s") * 2 + lax.axis_index("c")
    tokens_per = T // NW
    tbase = wid * tokens_per
    pltpu.sync_copy(pos0_h.at[pl.ds(tbase, tokens_per)], p_v.at[0])
    pltpu.sync_copy(pos1_h.at[pl.ds(tbase, tokens_per)], p_v.at[1])
    pltpu.sync_copy(x_h.at[pl.ds(tbase, tokens_per)], rows_v)
    c0 = pltpu.async_copy(rows_v, xs_h.at[p_v.at[0]], sem0)
    c1 = pltpu.async_copy(rows_v, xs_h.at[p_v.at[1]], sem1)
    c0.wait()
    c1.wait()


def _moe_routed_body(bexp_ref, nused_ref, xs_ref,
                     gq, gz, gs, uq, uz, us, dq, dz, ds,
                     ys_ref, wg_s, wu_s, wd_s):
    b = pl.program_id(0)
    e = bexp_ref[b]
    prev = bexp_ref[jnp.maximum(b - 1, 0)]

    @pl.when((b == 0) | (e != prev))
    def _():
        _dequant_to(gq, gz, gs, wg_s)
        _dequant_to(uq, uz, us, wu_s)
        _dequant_to(dq, dz, ds, wd_s)

    @pl.when(b < nused_ref[0])
    def _():
        xb = xs_ref[...].astype(jnp.bfloat16)
        dims = (((1,), (1,)), ((), ()))
        g = lax.dot_general(xb, wg_s[...], dims, preferred_element_type=jnp.float32)
        u = lax.dot_general(xb, wu_s[...], dims, preferred_element_type=jnp.float32)
        h = (g * jax.nn.sigmoid(g) * u).astype(jnp.bfloat16)
        ys_ref[...] = lax.dot_general(h, wd_s[...], dims,
                                      preferred_element_type=jnp.float32)


def _sc_collect_body(pos0_h, pos1_h, ys_h, y0_h, y1_h,
                     p_v, b0, b1, sem0, sem1, *, T, D):
    wid = lax.axis_index("s") * 2 + lax.axis_index("c")
    tokens_per = T // NW
    tbase = wid * tokens_per
    pltpu.sync_copy(pos0_h.at[pl.ds(tbase, tokens_per)], p_v.at[0])
    pltpu.sync_copy(pos1_h.at[pl.ds(tbase, tokens_per)], p_v.at[1])
    rows = b0.shape[0]
    for c in range(tokens_per // rows):
        c0 = pltpu.async_copy(ys_h.at[p_v.at[0, pl.ds(c * rows, rows)]], b0, sem0)
        c1 = pltpu.async_copy(ys_h.at[p_v.at[1, pl.ds(c * rows, rows)]], b1, sem1)
        c0.wait()
        pltpu.sync_copy(b0, y0_h.at[pl.ds(tbase + c * rows, rows)])
        c1.wait()
        pltpu.sync_copy(b1, y1_h.at[pl.ds(tbase + c * rows, rows)])


def _shared_body(xb_ref, y0_ref, y1_ref, w0_ref, w1_ref,
                 sg_ref, su_ref, sd_ref, out_ref):
    xb = xb_ref[...]
    dims = (((1,), (1,)), ((), ()))
    a = lax.dot_general(xb, sg_ref[...], dims, preferred_element_type=jnp.float32)
    b = lax.dot_general(xb, su_ref[...], dims, preferred_element_type=jnp.float32)
    h = (a * jax.nn.sigmoid(a) * b).astype(jnp.bfloat16)
    shared = lax.dot_general(h, sd_ref[...], dims,
                             preferred_element_type=jnp.float32)
    out_ref[...] = (shared + y0_ref[...] * w0_ref[...]
                    + y1_ref[...] * w1_ref[...])


def kernel(hidden_states, router_w, gate_q, gate_z, gate_s, up_q, up_z, up_s,
           down_q, down_z, down_s, shared_gate_w, shared_up_w, shared_down_w):
    B, S, D = hidden_states.shape
    T = B * S
    E, FF, _ = gate_q.shape
    n_g_in = gate_s.shape[-1]
    n_g_ff = down_s.shape[-1]
    SFF = shared_gate_w.shape[0]
    NB = (2 * T) // BLK + E
    P = NB * BLK

    x = hidden_states.reshape(T, D)
    xb = x.astype(jnp.bfloat16)
    gq8 = gate_q.astype(jnp.int8)
    gz8 = gate_z.astype(jnp.int8)
    uq8 = up_q.astype(jnp.int8)
    uz8 = up_z.astype(jnp.int8)
    dq8 = down_q.astype(jnp.int8)
    dz8 = down_z.astype(jnp.int8)
    sgb = shared_gate_w.astype(jnp.bfloat16)
    sub = shared_up_w.astype(jnp.bfloat16)
    sdb = shared_down_w.astype(jnp.bfloat16)

    pos0, pos1, w0, w1, bexp, nused = pl.pallas_call(
        functools.partial(_plan_body, nb=NB),
        grid=(1,),
        in_specs=[
            pl.BlockSpec((T, D), lambda i: (0, 0)),
            pl.BlockSpec((E, D), lambda i: (0, 0)),
        ],
        out_specs=[
            pl.BlockSpec((1, T), lambda i: (0, 0)),
            pl.BlockSpec((1, T), lambda i: (0, 0)),
            pl.BlockSpec((1, T), lambda i: (0, 0)),
            pl.BlockSpec((1, T), lambda i: (0, 0)),
            pl.BlockSpec((1, NB), lambda i: (0, 0)),
            pl.BlockSpec((1, 1), lambda i: (0, 0)),
        ],
        out_shape=[
            jax.ShapeDtypeStruct((1, T), jnp.int32),
            jax.ShapeDtypeStruct((1, T), jnp.int32),
            jax.ShapeDtypeStruct((1, T), jnp.float32),
            jax.ShapeDtypeStruct((1, T), jnp.float32),
            jax.ShapeDtypeStruct((1, NB), jnp.int32),
            jax.ShapeDtypeStruct((1, 1), jnp.int32),
        ],
    )(x, router_w)
    pos0f = pos0.reshape(T)
    pos1f = pos1.reshape(T)
    w0f = w0.reshape(T)
    w1f = w1.reshape(T)

    mesh = plsc.VectorSubcoreMesh(core_axis_name="c", subcore_axis_name="s")
    xs = pl.kernel(
        functools.partial(_sc_dispatch_body, T=T, P=P),
        mesh=mesh,
        out_type=jax.ShapeDtypeStruct((P, D), jnp.float32),
        scratch_types=[
            pltpu.VMEM((2, T // NW), jnp.int32),
            pltpu.VMEM((T // NW, D), jnp.float32),
            pltpu.SemaphoreType.DMA,
            pltpu.SemaphoreType.DMA,
        ],
    )(pos0f, pos1f, x)

    w_spec = lambda shape: pl.BlockSpec(
        (1,) + shape, lambda b, be, nu: (be[b], 0, 0))
    ys = pl.pallas_call(
        _moe_routed_body,
        grid_spec=pltpu.PrefetchScalarGridSpec(
            num_scalar_prefetch=2,
            grid=(NB,),
            in_specs=[
                pl.BlockSpec((BLK, D), lambda b, be, nu: (b, 0)),
                w_spec((FF, D)), w_spec((FF, n_g_in)), w_spec((FF, n_g_in)),
                w_spec((FF, D)), w_spec((FF, n_g_in)), w_spec((FF, n_g_in)),
                w_spec((D, FF)), w_spec((D, n_g_ff)), w_spec((D, n_g_ff)),
            ],
            out_specs=pl.BlockSpec((BLK, D), lambda b, be, nu: (b, 0)),
            scratch_shapes=[
                pltpu.VMEM((FF, D), jnp.bfloat16),
                pltpu.VMEM((FF, D), jnp.bfloat16),
                pltpu.VMEM((D, FF), jnp.bfloat16),
            ],
        ),
        out_shape=jax.ShapeDtypeStruct((P, D), jnp.float32),
    )(bexp.reshape(NB), nused.reshape(1), xs,
      gq8, gz8, gate_s, uq8, uz8, up_s, dq8, dz8, down_s)

    y0, y1 = pl.kernel(
        functools.partial(_sc_collect_body, T=T, D=D),
        mesh=mesh,
        out_type=[jax.ShapeDtypeStruct((T, D), jnp.float32),
                  jax.ShapeDtypeStruct((T, D), jnp.float32)],
        scratch_types=[
            pltpu.VMEM((2, T // NW), jnp.int32),
            pltpu.VMEM((T // NW // 2, D), jnp.float32),
            pltpu.VMEM((T // NW // 2, D), jnp.float32),
            pltpu.SemaphoreType.DMA,
            pltpu.SemaphoreType.DMA,
        ],
    )(pos0f, pos1f, ys)

    blk_s = min(T, 256)
    out = pl.pallas_call(
        _shared_body,
        grid=(T // blk_s,),
        in_specs=[
            pl.BlockSpec((blk_s, D), lambda m: (m, 0)),
            pl.BlockSpec((blk_s, D), lambda m: (m, 0)),
            pl.BlockSpec((blk_s, D), lambda m: (m, 0)),
            pl.BlockSpec((blk_s, 1), lambda m: (m, 0)),
            pl.BlockSpec((blk_s, 1), lambda m: (m, 0)),
            pl.BlockSpec((SFF, D), lambda m: (0, 0)),
            pl.BlockSpec((SFF, D), lambda m: (0, 0)),
            pl.BlockSpec((D, SFF), lambda m: (0, 0)),
        ],
        out_specs=pl.BlockSpec((blk_s, D), lambda m: (m, 0)),
        out_shape=jax.ShapeDtypeStruct((T, D), jnp.float32),
    )(xb, y0, y1, w0f.reshape(T, 1), w1f.reshape(T, 1), sgb, sub, sdb)

    return out.reshape(B, S, D)


# speedup vs baseline: 1.0493x; 1.0022x over previous
import functools

import jax
import jax.numpy as jnp
from jax import lax
from jax.experimental import pallas as pl
from jax.experimental.pallas import tpu as pltpu
from jax.experimental.pallas import tpu_sc as plsc

BLK = 512
NW = 32


def _group_selector(n_groups, in_dim, dtype):
    gsz = in_dim // n_groups
    col_g = lax.broadcasted_iota(jnp.int32, (n_groups, in_dim), 1) // gsz
    row_g = lax.broadcasted_iota(jnp.int32, (n_groups, in_dim), 0)
    return (col_g == row_g).astype(dtype)


def _dequant_to(q_ref, z_ref, s_ref, w_ref):
    O, I = w_ref.shape
    n_g = s_ref.shape[-1]
    sel = _group_selector(n_g, I, jnp.bfloat16)
    s = s_ref[0]
    zs = z_ref[0].astype(jnp.float32) * s
    dims = (((1,), (0,)), ((), ()))
    s_full = lax.dot_general(s.astype(jnp.bfloat16), sel, dims,
                             preferred_element_type=jnp.float32)
    zs_full = lax.dot_general(zs.astype(jnp.bfloat16), sel, dims,
                              preferred_element_type=jnp.float32)
    w = q_ref[0].astype(jnp.float32) * s_full - zs_full
    w_ref[...] = w.astype(jnp.bfloat16)


def _plan_body(x_ref, rw_ref, pos0_ref, pos1_ref, w0_ref, w1_ref,
               bexp_ref, nused_ref, *, nb):
    E, T = rw_ref.shape[0], x_ref.shape[0]
    logits = lax.dot_general(rw_ref[...], x_ref[...],
                             (((1,), (1,)), ((), ())),
                             preferred_element_type=jnp.float32)
    s = jax.nn.sigmoid(logits)
    riota = lax.broadcasted_iota(jnp.int32, s.shape, 0)
    i1 = jnp.argmax(s, axis=0)
    m1 = jnp.max(s, axis=0)
    oh1 = riota == i1[None, :]
    s2 = jnp.where(oh1, -1.0, s)
    i2 = jnp.argmax(s2, axis=0)
    m2 = jnp.max(s2, axis=0)
    oh2 = riota == i2[None, :]
    denom = m1 + m2
    w0_ref[...] = (m1 / denom)[None, :]
    w1_ref[...] = (m2 / denom)[None, :]

    r_t = lax.broadcasted_iota(jnp.int32, (T, T), 0)
    c_t = lax.broadcasted_iota(jnp.int32, (T, T), 1)
    tri = (r_t <= c_t).astype(jnp.bfloat16)
    dims = (((1,), (0,)), ((), ()))
    cum1 = lax.dot_general(oh1.astype(jnp.bfloat16), tri, dims,
                           preferred_element_type=jnp.float32)
    cum2 = lax.dot_general(oh2.astype(jnp.bfloat16), tri, dims,
                           preferred_element_type=jnp.float32)
    cnt1 = cum1[:, T - 1:T]
    counts = cnt1 + cum2[:, T - 1:T]
    padded = jnp.floor((counts + (BLK - 1)) * (1.0 / BLK)) * BLK
    r_e = lax.broadcasted_iota(jnp.int32, (E, E), 0)
    c_e = lax.broadcasted_iota(jnp.int32, (E, E), 1)
    sl = (c_e < r_e).astype(jnp.float32)
    seg_base = lax.dot_general(sl, padded, dims,
                               precision=lax.Precision.HIGHEST,
                               preferred_element_type=jnp.float32)
    pos0 = jnp.sum(jnp.where(oh1, seg_base + cum1 - 1.0, 0.0), axis=0)
    pos1 = jnp.sum(jnp.where(oh2, seg_base + cnt1 + cum2 - 1.0, 0.0), axis=0)
    pos0_ref[...] = pos0[None, :].astype(jnp.int32)
    pos1_ref[...] = pos1[None, :].astype(jnp.int32)

    bstart = (lax.broadcasted_iota(jnp.int32, (E, nb), 1) * BLK).astype(jnp.float32)
    bexp = jnp.sum((seg_base <= bstart).astype(jnp.int32), axis=0) - 1
    bexp_ref[...] = bexp[None, :]
    total = seg_base[E - 1:E, :] + padded[E - 1:E, :]
    nused_ref[...] = (total * (1.0 / BLK)).astype(jnp.int32)


def _sc_dispatch_body(pos0_h, pos1_h, x_h, xs_h, p_v, rows_v, sem0, sem1,
                      *, T, P):
    wid = lax.axis_index("s") * 2 + lax.axis_index("c")
    tokens_per = T // NW
    tbase = wid * tokens_per
    pltpu.sync_copy(pos0_h.at[pl.ds(tbase, tokens_per)], p_v.at[0])
    pltpu.sync_copy(pos1_h.at[pl.ds(tbase, tokens_per)], p_v.at[1])
    pltpu.sync_copy(x_h.at[pl.ds(tbase, tokens_per)], rows_v)
    c0 = pltpu.async_copy(rows_v, xs_h.at[p_v.at[0]], sem0)
    c1 = pltpu.async_copy(rows_v, xs_h.at[p_v.at[1]], sem1)
    c0.wait()
    c1.wait()


def _moe_routed_body(bexp_ref, nused_ref, xs_ref,
                     gq, gz, gs, uq, uz, us, dq, dz, ds,
                     ys_ref, wg_s, wu_s, wd_s):
    b = pl.program_id(0)
    e = bexp_ref[b]
    prev = bexp_ref[jnp.maximum(b - 1, 0)]

    @pl.when((b == 0) | (e != prev))
    def _():
        _dequant_to(gq, gz, gs, wg_s)
        _dequant_to(uq, uz, us, wu_s)
        _dequant_to(dq, dz, ds, wd_s)

    @pl.when(b < nused_ref[0])
    def _():
        xb = xs_ref[...].astype(jnp.bfloat16)
        dims = (((1,), (1,)), ((), ()))
        g = lax.dot_general(xb, wg_s[...], dims, preferred_element_type=jnp.float32)
        u = lax.dot_general(xb, wu_s[...], dims, preferred_element_type=jnp.float32)
        h = (g * jax.nn.sigmoid(g) * u).astype(jnp.bfloat16)
        ys_ref[...] = lax.dot_general(h, wd_s[...], dims,
                                      preferred_element_type=jnp.float32)


def _sc_collect_body(pos0_h, pos1_h, ys_h, y0_h, y1_h,
                     p_v, b0, b1, sem0, sem1, *, T, D):
    wid = lax.axis_index("s") * 2 + lax.axis_index("c")
    tokens_per = T // NW
    tbase = wid * tokens_per
    pltpu.sync_copy(pos0_h.at[pl.ds(tbase, tokens_per)], p_v.at[0])
    pltpu.sync_copy(pos1_h.at[pl.ds(tbase, tokens_per)], p_v.at[1])
    rows = b0.shape[0]
    for c in range(tokens_per // rows):
        c0 = pltpu.async_copy(ys_h.at[p_v.at[0, pl.ds(c * rows, rows)]], b0, sem0)
        c1 = pltpu.async_copy(ys_h.at[p_v.at[1, pl.ds(c * rows, rows)]], b1, sem1)
        c0.wait()
        pltpu.sync_copy(b0, y0_h.at[pl.ds(tbase + c * rows, rows)])
        c1.wait()
        pltpu.sync_copy(b1, y1_h.at[pl.ds(tbase + c * rows, rows)])


def _shared_body(xb_ref, y0_ref, y1_ref, w0_ref, w1_ref,
                 sg_ref, su_ref, sd_ref, out_ref):
    xb = xb_ref[...]
    dims = (((1,), (1,)), ((), ()))
    a = lax.dot_general(xb, sg_ref[...], dims, preferred_element_type=jnp.float32)
    b = lax.dot_general(xb, su_ref[...], dims, preferred_element_type=jnp.float32)
    h = (a * jax.nn.sigmoid(a) * b).astype(jnp.bfloat16)
    shared = lax.dot_general(h, sd_ref[...], dims,
                             preferred_element_type=jnp.float32)
    out_ref[...] = (shared + y0_ref[...] * w0_ref[...]
                    + y1_ref[...] * w1_ref[...])


def kernel(hidden_states, router_w, gate_q, gate_z, gate_s, up_q, up_z, up_s,
           down_q, down_z, down_s, shared_gate_w, shared_up_w, shared_down_w):
    B, S, D = hidden_states.shape
    T = B * S
    E, FF, _ = gate_q.shape
    n_g_in = gate_s.shape[-1]
    n_g_ff = down_s.shape[-1]
    SFF = shared_gate_w.shape[0]
    NB = (2 * T) // BLK + E
    P = NB * BLK

    x = hidden_states.reshape(T, D)
    xb = x.astype(jnp.bfloat16)
    gq8 = gate_q.astype(jnp.int8)
    gz8 = gate_z.astype(jnp.int8)
    uq8 = up_q.astype(jnp.int8)
    uz8 = up_z.astype(jnp.int8)
    dq8 = down_q.astype(jnp.int8)
    dz8 = down_z.astype(jnp.int8)
    sgb = shared_gate_w.astype(jnp.bfloat16)
    sub = shared_up_w.astype(jnp.bfloat16)
    sdb = shared_down_w.astype(jnp.bfloat16)

    pos0, pos1, w0, w1, bexp, nused = pl.pallas_call(
        functools.partial(_plan_body, nb=NB),
        grid=(1,),
        in_specs=[
            pl.BlockSpec((T, D), lambda i: (0, 0)),
            pl.BlockSpec((E, D), lambda i: (0, 0)),
        ],
        out_specs=[
            pl.BlockSpec((1, T), lambda i: (0, 0)),
            pl.BlockSpec((1, T), lambda i: (0, 0)),
            pl.BlockSpec((1, T), lambda i: (0, 0)),
            pl.BlockSpec((1, T), lambda i: (0, 0)),
            pl.BlockSpec((1, NB), lambda i: (0, 0)),
            pl.BlockSpec((1, 1), lambda i: (0, 0)),
        ],
        out_shape=[
            jax.ShapeDtypeStruct((1, T), jnp.int32),
            jax.ShapeDtypeStruct((1, T), jnp.int32),
            jax.ShapeDtypeStruct((1, T), jnp.float32),
            jax.ShapeDtypeStruct((1, T), jnp.float32),
            jax.ShapeDtypeStruct((1, NB), jnp.int32),
            jax.ShapeDtypeStruct((1, 1), jnp.int32),
        ],
    )(x, router_w)
    pos0f = pos0.reshape(T)
    pos1f = pos1.reshape(T)
    w0f = w0.reshape(T)
    w1f = w1.reshape(T)

    mesh = plsc.VectorSubcoreMesh(core_axis_name="c", subcore_axis_name="s")
    xs = pl.kernel(
        functools.partial(_sc_dispatch_body, T=T, P=P),
        mesh=mesh,
        out_type=jax.ShapeDtypeStruct((P, D), jnp.float32),
        scratch_types=[
            pltpu.VMEM((2, T // NW), jnp.int32),
            pltpu.VMEM((T // NW, D), jnp.float32),
            pltpu.SemaphoreType.DMA,
            pltpu.SemaphoreType.DMA,
        ],
    )(pos0f, pos1f, x)

    w_spec = lambda shape: pl.BlockSpec(
        (1,) + shape, lambda b, be, nu: (be[b], 0, 0))
    ys = pl.pallas_call(
        _moe_routed_body,
        grid_spec=pltpu.PrefetchScalarGridSpec(
            num_scalar_prefetch=2,
            grid=(NB,),
            in_specs=[
                pl.BlockSpec((BLK, D), lambda b, be, nu: (b, 0)),
                w_spec((FF, D)), w_spec((FF, n_g_in)), w_spec((FF, n_g_in)),
                w_spec((FF, D)), w_spec((FF, n_g_in)), w_spec((FF, n_g_in)),
                w_spec((D, FF)), w_spec((D, n_g_ff)), w_spec((D, n_g_ff)),
            ],
            out_specs=pl.BlockSpec((BLK, D), lambda b, be, nu: (b, 0)),
            scratch_shapes=[
                pltpu.VMEM((FF, D), jnp.bfloat16),
                pltpu.VMEM((FF, D), jnp.bfloat16),
                pltpu.VMEM((D, FF), jnp.bfloat16),
            ],
        ),
        out_shape=jax.ShapeDtypeStruct((P, D), jnp.float32),
    )(bexp.reshape(NB), nused.reshape(1), xs,
      gq8, gz8, gate_s, uq8, uz8, up_s, dq8, dz8, down_s)

    y0, y1 = pl.kernel(
        functools.partial(_sc_collect_body, T=T, D=D),
        mesh=mesh,
        out_type=[jax.ShapeDtypeStruct((T, D), jnp.float32),
                  jax.ShapeDtypeStruct((T, D), jnp.float32)],
        scratch_types=[
            pltpu.VMEM((2, T // NW), jnp.int32),
            pltpu.VMEM((T // NW // 2, D), jnp.float32),
            pltpu.VMEM((T // NW // 2, D), jnp.float32),
            pltpu.SemaphoreType.DMA,
            pltpu.SemaphoreType.DMA,
        ],
    )(pos0f, pos1f, ys)

    blk_s = min(T, 256)
    out = pl.pallas_call(
        _shared_body,
        grid=(T // blk_s,),
        in_specs=[
            pl.BlockSpec((blk_s, D), lambda m: (m, 0)),
            pl.BlockSpec((blk_s, D), lambda m: (m, 0)),
            pl.BlockSpec((blk_s, D), lambda m: (m, 0)),
            pl.BlockSpec((blk_s, 1), lambda m: (m, 0)),
            pl.BlockSpec((blk_s, 1), lambda m: (m, 0)),
            pl.BlockSpec((SFF, D), lambda m: (0, 0)),
            pl.BlockSpec((SFF, D), lambda m: (0, 0)),
            pl.BlockSpec((D, SFF), lambda m: (0, 0)),
        ],
        out_specs=pl.BlockSpec((blk_s, D), lambda m: (m, 0)),
        out_shape=jax.ShapeDtypeStruct((T, D), jnp.float32),
    )(xb, y0, y1, w0f.reshape(T, 1), w1f.reshape(T, 1), sgb, sub, sdb)

    return out.reshape(B, S, D)
